# baseline (speedup 1.0000x reference)
import functools

import jax
import jax.numpy as jnp
from jax import lax
from jax.experimental import pallas as pl
from jax.experimental.pallas import tpu as pltpu
from jax.experimental.pallas import tpu_sc as plsc

B = 16384
F = 512
DG = 32
NC = 2
NS = 16
NW = NC * NS
BPW = B // NW
CH = 64
NCHUNK = BPW // CH
SCH = 128
NSCHUNK = BPW // SCH


def _sc_gather_body(ui_hbm, pi_hbm, ni_hbm, features, gamma_users, gamma_items,
                    theta_users, beta_items,
                    pf_out, nf_out, gu_out, gip_out, gin_out, tu_out, bp_out, bn_out,
                    ui_v, pi_v, ni_v, pf_v, nf_v, gu_v, gip_v, gin_v, tu_v,
                    bp_v, bn_v, sem):
    wid = lax.axis_index("c") * NS + lax.axis_index("s")
    base = wid * BPW

    pltpu.sync_copy(ui_hbm.at[pl.ds(base, BPW)], ui_v)
    pltpu.sync_copy(pi_hbm.at[pl.ds(base, BPW)], pi_v)
    pltpu.sync_copy(ni_hbm.at[pl.ds(base, BPW)], ni_v)

    for c in range(NSCHUNK):
        off = c * SCH
        ui_c = ui_v.at[pl.ds(off, SCH)]
        pi_c = pi_v.at[pl.ds(off, SCH)]
        ni_c = ni_v.at[pl.ds(off, SCH)]
        copies = [
            pltpu.async_copy(gamma_users.at[ui_c], gu_v, sem),
            pltpu.async_copy(theta_users.at[ui_c], tu_v, sem),
            pltpu.async_copy(gamma_items.at[pi_c], gip_v, sem),
            pltpu.async_copy(gamma_items.at[ni_c], gin_v, sem),
            pltpu.async_copy(beta_items.at[pi_c], bp_v, sem),
            pltpu.async_copy(beta_items.at[ni_c], bn_v, sem),
        ]
        for cp in copies:
            cp.wait()
        pltpu.sync_copy(gu_v, gu_out.at[pl.ds(base + off, SCH)])
        pltpu.sync_copy(tu_v, tu_out.at[pl.ds(base + off, SCH)])
        pltpu.sync_copy(gip_v, gip_out.at[pl.ds(base + off, SCH)])
        pltpu.sync_copy(gin_v, gin_out.at[pl.ds(base + off, SCH)])
        pltpu.sync_copy(bp_v, bp_out.at[pl.ds(base + off, SCH)])
        pltpu.sync_copy(bn_v, bn_out.at[pl.ds(base + off, SCH)])

    for c in range(NCHUNK):
        off = c * CH
        cp1 = pltpu.async_copy(features.at[pi_v.at[pl.ds(off, CH)]], pf_v, sem)
        cp2 = pltpu.async_copy(features.at[ni_v.at[pl.ds(off, CH)]], nf_v, sem)
        cp1.wait()
        cp2.wait()
        pltpu.sync_copy(pf_v, pf_out.at[pl.ds(base + off, CH)])
        pltpu.sync_copy(nf_v, nf_out.at[pl.ds(base + off, CH)])


@functools.partial(
    pl.kernel,
    out_type=(
        jax.ShapeDtypeStruct((B, F), jnp.float32),
        jax.ShapeDtypeStruct((B, F), jnp.float32),
        jax.ShapeDtypeStruct((B, DG), jnp.float32),
        jax.ShapeDtypeStruct((B, DG), jnp.float32),
        jax.ShapeDtypeStruct((B, DG), jnp.float32),
        jax.ShapeDtypeStruct((B, DG), jnp.float32),
        jax.ShapeDtypeStruct((B, 1), jnp.float32),
        jax.ShapeDtypeStruct((B, 1), jnp.float32),
    ),
    mesh=plsc.VectorSubcoreMesh(core_axis_name="c", subcore_axis_name="s"),
    compiler_params=pltpu.CompilerParams(use_tc_tiling_on_sc=False),
    scratch_types=[
        pltpu.VMEM((BPW,), jnp.int32),
        pltpu.VMEM((BPW,), jnp.int32),
        pltpu.VMEM((BPW,), jnp.int32),
        pltpu.VMEM((CH, F), jnp.float32),
        pltpu.VMEM((CH, F), jnp.float32),
        pltpu.VMEM((SCH, DG), jnp.float32),
        pltpu.VMEM((SCH, DG), jnp.float32),
        pltpu.VMEM((SCH, DG), jnp.float32),
        pltpu.VMEM((SCH, DG), jnp.float32),
        pltpu.VMEM((SCH, 1), jnp.float32),
        pltpu.VMEM((SCH, 1), jnp.float32),
        pltpu.SemaphoreType.DMA,
    ],
)
def _sc_gather(*refs):
    _sc_gather_body(*refs)


BB = 1024


def _tc_combine_body(pf, nf, gu, gip, gin, tu, bp, bn, emb, vb, out):
    diff = pf[...] - nf[...]
    g = jnp.dot(diff, emb[...], preferred_element_type=jnp.float32)
    s_vis = jnp.sum(tu[...] * g, axis=1)
    s_bias = jnp.dot(diff, vb[...], preferred_element_type=jnp.float32)[:, 0]
    s_lat = jnp.sum(gu[...] * (gip[...] - gin[...]), axis=1)
    out[...] = bp[...][:, 0] - bn[...][:, 0] + s_lat + s_vis + s_bias


def _tc_combine(pf, nf, gu, gip, gin, tu, bp, bn, emb, vb):
    grid = (B // BB,)
    return pl.pallas_call(
        _tc_combine_body,
        grid=grid,
        in_specs=[
            pl.BlockSpec((BB, F), lambda i: (i, 0)),
            pl.BlockSpec((BB, F), lambda i: (i, 0)),
            pl.BlockSpec((BB, DG), lambda i: (i, 0)),
            pl.BlockSpec((BB, DG), lambda i: (i, 0)),
            pl.BlockSpec((BB, DG), lambda i: (i, 0)),
            pl.BlockSpec((BB, DG), lambda i: (i, 0)),
            pl.BlockSpec((BB, 1), lambda i: (i, 0)),
            pl.BlockSpec((BB, 1), lambda i: (i, 0)),
            pl.BlockSpec((F, DG), lambda i: (0, 0)),
            pl.BlockSpec((F, 1), lambda i: (0, 0)),
        ],
        out_specs=pl.BlockSpec((BB,), lambda i: (i,)),
        out_shape=jax.ShapeDtypeStruct((B,), jnp.float32),
    )(pf, nf, gu, gip, gin, tu, bp, bn, emb, vb)


def kernel(ui, pi, ni, features, gamma_users, gamma_items, theta_users,
           embedding, beta_items, visual_bias):
    pf, nf, gu, gip, gin, tu, bp, bn = _sc_gather(
        ui, pi, ni, features, gamma_users, gamma_items, theta_users, beta_items)
    pf = jnp.take(features, pi, axis=0)
    nf = jnp.take(features, ni, axis=0)
    return _tc_combine(pf, nf, gu, gip, gin, tu, bp, bn, embedding, visual_bias)


# speedup vs baseline: 1.5524x; 1.5524x over previous
import functools

import jax
import jax.numpy as jnp
from jax import lax
from jax.experimental import pallas as pl
from jax.experimental.pallas import tpu as pltpu
from jax.experimental.pallas import tpu_sc as plsc

B = 16384
F = 512
DG = 32
LANES = 16
NC = 2
NS = 16
NW = NC * NS
BPW = B // NW
CH = 32
NCHUNK = BPW // CH
SCH = 64
NSCHUNK = BPW // SCH
NBETA = 100096 // 128


def _shift_into(src_v, dst_v, sh):
    for k in range(BPW // LANES):
        sl = pl.ds(k * LANES, LANES)
        dst_v[sl] = lax.shift_right_logical(src_v[sl], sh)


def _sc_gather_body(ui_hbm, pi_hbm, ni_hbm, features, gu128, tu128, gi128, beta128,
                    pf_out, nf_out, gur_out, tur_out, gipr_out, ginr_out,
                    bpr_out, bnr_out,
                    ui_v, pi_v, ni_v, ui2_v, pi2_v, ni2_v, pi7_v, ni7_v,
                    pf_v, nf_v, gur_v, tur_v, gipr_v, ginr_v, bpr_v, bnr_v, sem):
    wid = lax.axis_index("c") * NS + lax.axis_index("s")
    base = wid * BPW

    pltpu.sync_copy(ui_hbm.at[pl.ds(base, BPW)], ui_v)
    pltpu.sync_copy(pi_hbm.at[pl.ds(base, BPW)], pi_v)
    pltpu.sync_copy(ni_hbm.at[pl.ds(base, BPW)], ni_v)

    _shift_into(ui_v, ui2_v, 2)
    _shift_into(pi_v, pi2_v, 2)
    _shift_into(ni_v, ni2_v, 2)
    _shift_into(pi_v, pi7_v, 7)
    _shift_into(ni_v, ni7_v, 7)

    for c in range(NSCHUNK):
        off = c * SCH
        sl = pl.ds(off, SCH)
        copies = [
            pltpu.async_copy(gu128.at[ui2_v.at[sl]], gur_v, sem),
            pltpu.async_copy(tu128.at[ui2_v.at[sl]], tur_v, sem),
            pltpu.async_copy(gi128.at[pi2_v.at[sl]], gipr_v, sem),
            pltpu.async_copy(gi128.at[ni2_v.at[sl]], ginr_v, sem),
            pltpu.async_copy(beta128.at[pi7_v.at[sl]], bpr_v, sem),
            pltpu.async_copy(beta128.at[ni7_v.at[sl]], bnr_v, sem),
        ]
        for cp in copies:
            cp.wait()
        osl = pl.ds(base + off, SCH)
        pltpu.sync_copy(gur_v, gur_out.at[osl])
        pltpu.sync_copy(tur_v, tur_out.at[osl])
        pltpu.sync_copy(gipr_v, gipr_out.at[osl])
        pltpu.sync_copy(ginr_v, ginr_out.at[osl])
        pltpu.sync_copy(bpr_v, bpr_out.at[osl])
        pltpu.sync_copy(bnr_v, bnr_out.at[osl])

    for c in range(NCHUNK):
        off = c * CH
        cp1 = pltpu.async_copy(features.at[pi_v.at[pl.ds(off, CH)]], pf_v, sem)
        cp2 = pltpu.async_copy(features.at[ni_v.at[pl.ds(off, CH)]], nf_v, sem)
        cp1.wait()
        cp2.wait()
        pltpu.sync_copy(pf_v, pf_out.at[pl.ds(base + off, CH)])
        pltpu.sync_copy(nf_v, nf_out.at[pl.ds(base + off, CH)])


_scratch = [
    pltpu.VMEM((BPW,), jnp.int32),
    pltpu.VMEM((BPW,), jnp.int32),
    pltpu.VMEM((BPW,), jnp.int32),
    pltpu.VMEM((BPW,), jnp.int32),
    pltpu.VMEM((BPW,), jnp.int32),
    pltpu.VMEM((BPW,), jnp.int32),
    pltpu.VMEM((BPW,), jnp.int32),
    pltpu.VMEM((BPW,), jnp.int32),
    pltpu.VMEM((CH, F), jnp.float32),
    pltpu.VMEM((CH, F), jnp.float32),
    pltpu.VMEM((SCH, 128), jnp.float32),
    pltpu.VMEM((SCH, 128), jnp.float32),
    pltpu.VMEM((SCH, 128), jnp.float32),
    pltpu.VMEM((SCH, 128), jnp.float32),
    pltpu.VMEM((SCH, 128), jnp.float32),
    pltpu.VMEM((SCH, 128), jnp.float32),
    pltpu.SemaphoreType.DMA,
]


@functools.partial(
    pl.kernel,
    out_type=(
        jax.ShapeDtypeStruct((B, F), jnp.float32),
        jax.ShapeDtypeStruct((B, F), jnp.float32),
        jax.ShapeDtypeStruct((B, 128), jnp.float32),
        jax.ShapeDtypeStruct((B, 128), jnp.float32),
        jax.ShapeDtypeStruct((B, 128), jnp.float32),
        jax.ShapeDtypeStruct((B, 128), jnp.float32),
        jax.ShapeDtypeStruct((B, 128), jnp.float32),
        jax.ShapeDtypeStruct((B, 128), jnp.float32),
    ),
    mesh=plsc.VectorSubcoreMesh(core_axis_name="c", subcore_axis_name="s"),
    scratch_types=_scratch,
)
def _sc_gather(*refs):
    _sc_gather_body(*refs)


BB = 1024


def _sel32(row128, off):
    out = jnp.zeros((row128.shape[0], DG), jnp.float32)
    for j in range(4):
        out = out + jnp.where(off == j, row128[:, j * DG:(j + 1) * DG], 0.0)
    return out


def _tc_combine_body(ui, pi, ni, pf, nf, gur, tur, gipr, ginr, bpr, bnr,
                     emb, vb, out):
    uii, pii, nii = ui[...], pi[...], ni[...]
    uoff = lax.rem(uii, 4)
    gu = _sel32(gur[...], uoff)
    tu = _sel32(tur[...], uoff)
    gip = _sel32(gipr[...], lax.rem(pii, 4))
    gin = _sel32(ginr[...], lax.rem(nii, 4))
    col = jax.lax.broadcasted_iota(jnp.int32, (BB, 128), 1)
    bp = jnp.sum(jnp.where(col == lax.rem(pii, 128), bpr[...], 0.0), axis=1,
                 keepdims=True)
    bn = jnp.sum(jnp.where(col == lax.rem(nii, 128), bnr[...], 0.0), axis=1,
                 keepdims=True)
    diff = pf[...] - nf[...]
    g = jnp.dot(diff, emb[...], preferred_element_type=jnp.float32,
                precision=lax.Precision.HIGHEST)
    s_vis = jnp.sum(tu * g, axis=1, keepdims=True)
    s_bias = jnp.dot(diff, vb[...], preferred_element_type=jnp.float32,
                     precision=lax.Precision.HIGHEST)
    s_lat = jnp.sum(gu * (gip - gin), axis=1, keepdims=True)
    out[...] = bp - bn + s_lat + s_vis + s_bias


def _tc_combine(ui, pi, ni, pf, nf, gur, tur, gipr, ginr, bpr, bnr, emb, vb):
    bspec_b = pl.BlockSpec((BB, 1), lambda i: (i, 0))
    bspec_f = pl.BlockSpec((BB, F), lambda i: (i, 0))
    bspec_s = pl.BlockSpec((BB, 128), lambda i: (i, 0))
    return pl.pallas_call(
        _tc_combine_body,
        grid=(B // BB,),
        in_specs=[
            bspec_b, bspec_b, bspec_b,
            bspec_f, bspec_f,
            bspec_s, bspec_s, bspec_s, bspec_s, bspec_s, bspec_s,
            pl.BlockSpec((F, DG), lambda i: (0, 0)),
            pl.BlockSpec((F, 1), lambda i: (0, 0)),
        ],
        out_specs=bspec_b,
        out_shape=jax.ShapeDtypeStruct((B, 1), jnp.float32),
    )(ui.reshape(B, 1), pi.reshape(B, 1), ni.reshape(B, 1),
      pf, nf, gur, tur, gipr, ginr, bpr, bnr, emb, vb)[:, 0]


def kernel(ui, pi, ni, features, gamma_users, gamma_items, theta_users,
           embedding, beta_items, visual_bias):
    gu128 = gamma_users.reshape(-1, 128)
    tu128 = theta_users.reshape(-1, 128)
    gi128 = gamma_items.reshape(-1, 128)
    beta128 = jnp.pad(beta_items.reshape(-1), (0, NBETA * 128 - 100000)).reshape(NBETA, 128)
    pf, nf, gur, tur, gipr, ginr, bpr, bnr = _sc_gather(
        ui, pi, ni, features, gu128, tu128, gi128, beta128)
    return _tc_combine(ui, pi, ni, pf, nf, gur, tur, gipr, ginr, bpr, bnr,
                       embedding, visual_bias)
